# Initial kernel scaffold; baseline (speedup 1.0000x reference)
import jax
import jax.numpy as jnp
from jax.experimental import pallas as pl


def kernel(x, edge_index0, pseudo0, edge_index1, pseudo1, W1, root1, b1, W2, root2, b2, W3, root3, b3, W4, root4, b4, fc1_w, fc1_b, fc2_w, fc2_b):
    raise NotImplementedError("write your pallas kernel here")


# speedup vs baseline: 1.9603x; 1.9603x over previous
import functools
import jax
import jax.numpy as jnp
from jax import lax
from jax.experimental import pallas as pl
from jax.experimental.pallas import tpu as pltpu
from jax.experimental.pallas import tpu_sc as plsc

NW = 32
N0, E0 = 65536, 1048576
N1, E1 = 32768, 524288
PAD = 1024

MESH = plsc.VectorSubcoreMesh(core_axis_name="c", subcore_axis_name="s")
SC_PARAMS = pltpu.CompilerParams(needs_layout_passes=False)
IOTA = lambda: lax.iota(jnp.int32, 16)


def _wid():
  return lax.axis_index("s") * 2 + lax.axis_index("c")


def _vfull_i(v):
  return jnp.full((16,), 1, jnp.int32) * v


def _vfull_f(v):
  return jnp.full((16,), 1.0, jnp.float32) * v


def _make_a1(E, SHB):
  C = 512
  NCH = E // NW // C

  @functools.partial(
      pl.kernel,
      out_type=jax.ShapeDtypeStruct((NW, 32), jnp.int32),
      mesh=MESH,
      scratch_types=[
          pltpu.VMEM((C,), jnp.int32), pltpu.VMEM((C,), jnp.int32),
          pltpu.VMEM((32,), jnp.int32),
          pltpu.SemaphoreType.DMA, pltpu.SemaphoreType.DMA,
      ],
      compiler_params=SC_PARAMS,
  )
  def a1(dst_h, hist, b0, b1, vh, sr0, sr1):
    w = _wid()
    base = w * (E // NW)
    zero = jnp.zeros((16,), jnp.int32)
    vh[pl.ds(0, 16)] = zero
    vh[pl.ds(16, 16)] = zero
    ones = jnp.full((16,), 1, jnp.int32)

    def fire(j, buf, sem):
      jc = jnp.minimum(j, NCH - 1)
      p = pl.multiple_of(base + jc * C, 256)
      pltpu.make_async_copy(dst_h.at[pl.ds(p, C)], buf, sem).start()

    def wait(buf, sem):
      pltpu.make_async_copy(dst_h.at[pl.ds(base, C)], buf, sem).wait()

    fire(0, b0, sr0)
    fire(1, b1, sr1)

    def pair(t, carry):
      for jj, (buf, sem) in enumerate(((b0, sr0), (b1, sr1))):
        j = 2 * t + jj
        wait(buf, sem)
        for g in range(C // 16):
          d16 = buf[pl.ds(g * 16, 16)]
          bk = d16 >> SHB
          plsc.addupdate_scatter(vh, [bk], ones)
        fire(j + 2, buf, sem)
      return carry

    lax.fori_loop(0, NCH // 2, pair, 0)
    wait(b0, sr0)
    wait(b1, sr1)
    pltpu.sync_copy(vh, hist.at[w])

  return a1


def _make_a2(N, E, SHB, SHN):
  C = 256
  NCH = E // NW // C
  EP = E + PAD

  out_type = (
      jax.ShapeDtypeStruct((EP,), jnp.int32),
      jax.ShapeDtypeStruct((EP,), jnp.float32),
      jax.ShapeDtypeStruct((EP,), jnp.float32),
      jax.ShapeDtypeStruct((EP,), jnp.int32),
      jax.ShapeDtypeStruct((48,), jnp.int32),
  )
  scratch = [
      pltpu.VMEM((C,), jnp.int32), pltpu.VMEM((C,), jnp.int32),
      pltpu.VMEM((C,), jnp.int32), pltpu.VMEM((C,), jnp.int32),
      pltpu.VMEM((2 * C,), jnp.float32), pltpu.VMEM((2 * C,), jnp.float32),
      pltpu.VMEM((NW, 32), jnp.int32),
      pltpu.VMEM((32,), jnp.int32),
      pltpu.VMEM((32,), jnp.int32),
      pltpu.VMEM((16,), jnp.int32),
      pltpu.VMEM((C,), jnp.int32), pltpu.VMEM((C,), jnp.int32),
      pltpu.VMEM((C,), jnp.float32), pltpu.VMEM((C,), jnp.float32),
      pltpu.VMEM((C,), jnp.float32), pltpu.VMEM((C,), jnp.float32),
      pltpu.VMEM((C,), jnp.int32), pltpu.VMEM((C,), jnp.int32),
      pltpu.VMEM((48,), jnp.int32),
      pltpu.SemaphoreType.DMA, pltpu.SemaphoreType.DMA,
      pltpu.SemaphoreType.DMA, pltpu.SemaphoreType.DMA,
  ]

  @functools.partial(pl.kernel, out_type=out_type, mesh=MESH,
                     scratch_types=scratch, compiler_params=SC_PARAMS)
  def a2(src_h, dst_h, ps_h, hist, rsk, rf0, rf1, rdst, boff,
         bs0, bs1, bd0, bd1, bp0, bp1, vhist, vbase, vpre, vbk,
         tsk0, tsk1, tf00, tf01, tf10, tf11, td0, td1, tboff,
         sr0, sr1, ss0, ss1):
    w = _wid()
    ew = E // NW
    base = w * ew
    iota = IOTA()
    ones = jnp.full((16,), 1, jnp.int32)

    pltpu.sync_copy(hist, vhist)
    wv = _vfull_i(w)
    acc0 = jnp.zeros((16,), jnp.int32)
    acc1 = jnp.zeros((16,), jnp.int32)
    tot0 = jnp.zeros((16,), jnp.int32)
    tot1 = jnp.zeros((16,), jnp.int32)
    for wp in range(NW):
      r0 = vhist[wp, pl.ds(0, 16)]
      r1 = vhist[wp, pl.ds(16, 16)]
      m = jnp.where(jnp.full((16,), wp, jnp.int32) < wv, 1, 0)
      acc0 = acc0 + r0 * m
      acc1 = acc1 + r1 * m
      tot0 = tot0 + r0
      tot1 = tot1 + r1
    vpre[pl.ds(0, 16)] = tot0
    vpre[pl.ds(16, 16)] = tot1
    g0 = iota
    g1 = iota + 16
    for s in (1, 2, 4, 8, 16):
      a0 = vpre[pl.ds(0, 16)]
      a1v = vpre[pl.ds(16, 16)]
      s0 = plsc.load_gather(vpre, [jnp.maximum(g0 - s, 0)])
      s1 = plsc.load_gather(vpre, [jnp.maximum(g1 - s, 0)])
      a0 = a0 + jnp.where(g0 >= s, s0, 0)
      a1v = a1v + jnp.where(g1 >= s, s1, 0)
      vpre[pl.ds(0, 16)] = a0
      vpre[pl.ds(16, 16)] = a1v
    e0 = jnp.where(g0 >= 1, plsc.load_gather(vpre, [jnp.maximum(g0 - 1, 0)]), 0)
    e1 = plsc.load_gather(vpre, [g1 - 1])
    vbase[pl.ds(0, 16)] = e0 + acc0
    vbase[pl.ds(16, 16)] = e1 + acc1

    @pl.when(w == 0)
    def _():
      tboff[pl.ds(0, 16)] = e0
      tboff[pl.ds(16, 16)] = e1
      tboff[pl.ds(32, 16)] = jnp.full((16,), E, jnp.int32)
      pltpu.sync_copy(tboff, boff)

    def fire_rec(j, bs, bd, bp, sem):
      jc = jnp.minimum(j, NCH - 1)
      p = pl.multiple_of(base + jc * C, 256)
      pltpu.make_async_copy(src_h.at[pl.ds(p, C)], bs, sem).start()
      pltpu.make_async_copy(dst_h.at[pl.ds(p, C)], bd, sem).start()
      p2 = pl.multiple_of(2 * p, 512)
      pltpu.make_async_copy(ps_h.at[pl.ds(p2, 2 * C)], bp, sem).start()

    def wait_rec(bs, bd, bp, sem):
      pltpu.make_async_copy(src_h.at[pl.ds(base, C)], bs, sem).wait()
      pltpu.make_async_copy(dst_h.at[pl.ds(base, C)], bd, sem).wait()
      pltpu.make_async_copy(ps_h.at[pl.ds(base, 2 * C)], bp, sem).wait()

    def drain_scat(tsk, tf0, tf1, td, sem):
      pltpu.make_async_copy(rsk.at[pl.ds(0, C)], tsk, sem).wait()
      pltpu.make_async_copy(rf0.at[pl.ds(0, C)], tf0, sem).wait()
      pltpu.make_async_copy(rf1.at[pl.ds(0, C)], tf1, sem).wait()
      pltpu.make_async_copy(rdst.at[pl.ds(0, C)], td, sem).wait()

    sets = (
        (bs0, bd0, bp0, sr0, tsk0, tf00, tf10, td0, ss0),
        (bs1, bd1, bp1, sr1, tsk1, tf01, tf11, td1, ss1),
    )
    fire_rec(0, bs0, bd0, bp0, sr0)
    fire_rec(1, bs1, bd1, bp1, sr1)

    def pair(t, carry):
      for jj, (bs, bd, bp, srx, tsk, tf0, tf1, td, ssx) in enumerate(sets):
        j = 2 * t + jj

        @pl.when(j >= 2)
        def _():
          drain_scat(tsk, tf0, tf1, td, ssx)

        wait_rec(bs, bd, bp, srx)
        for g in range(C // 16):
          off = g * 16
          s16 = bs[pl.ds(off, 16)]
          d16 = bd[pl.ds(off, 16)]
          p0 = plsc.load_gather(bp, [(off + iota) * 2])
          p1 = plsc.load_gather(bp, [(off + iota) * 2 + 1])
          pos0 = p0 * 3.0
          pos1 = p1 * 3.0
          i0 = jnp.clip(pos0.astype(jnp.int32), 0, 2)
          i1 = jnp.clip(pos1.astype(jnp.int32), 0, 2)
          f0 = pos0 - i0.astype(jnp.float32)
          f1 = pos1 - i1.astype(jnp.float32)
          sk = s16 | (i0 << SHN) | (i1 << (SHN + 2))
          bk = d16 >> SHB
          vbk[...] = bk
          rank = jnp.zeros((16,), jnp.int32)
          for sft in range(1, 16):
            other = plsc.load_gather(vbk, [(iota - sft) & 15])
            hit = (other == bk) & (iota >= sft)
            rank = rank + jnp.where(hit, 1, 0)
          bb = plsc.load_gather(vbase, [bk])
          pos = bb + rank
          plsc.addupdate_scatter(vbase, [bk], ones)
          tsk[pl.ds(off, 16)] = sk
          tf0[pl.ds(off, 16)] = f0
          tf1[pl.ds(off, 16)] = f1
          td[pl.ds(off, 16)] = d16
          pltpu.make_async_copy(tsk.at[pl.ds(off, 16)], rsk.at[pos], ssx).start()
          pltpu.make_async_copy(tf0.at[pl.ds(off, 16)], rf0.at[pos], ssx).start()
          pltpu.make_async_copy(tf1.at[pl.ds(off, 16)], rf1.at[pos], ssx).start()
          pltpu.make_async_copy(td.at[pl.ds(off, 16)], rdst.at[pos], ssx).start()
        fire_rec(j + 2, bs, bd, bp, srx)
      return carry

    lax.fori_loop(0, NCH // 2, pair, 0)
    for (bs, bd, bp, srx, tsk, tf0, tf1, td, ssx) in sets:
      wait_rec(bs, bd, bp, srx)
      drain_scat(tsk, tf0, tf1, td, ssx)

  return a2


def _make_conv1(N, E, SHB, SHN):
  C = 256
  BS = N // NW
  EP = E + PAD

  out_type = (
      jax.ShapeDtypeStruct((N * 16,), jnp.float32),
      jax.ShapeDtypeStruct((N,), jnp.float32),
  )
  scratch = [
      pltpu.VMEM((N,), jnp.float32),
      pltpu.VMEM((BS * 16,), jnp.float32),
      pltpu.VMEM((BS,), jnp.float32),
      pltpu.VMEM((C,), jnp.int32), pltpu.VMEM((C,), jnp.int32),
      pltpu.VMEM((C,), jnp.float32), pltpu.VMEM((C,), jnp.float32),
      pltpu.VMEM((C,), jnp.float32), pltpu.VMEM((C,), jnp.float32),
      pltpu.VMEM((C,), jnp.int32), pltpu.VMEM((C,), jnp.int32),
      pltpu.VMEM((48,), jnp.int32),
      pltpu.VMEM_SHARED((48,), jnp.int32),
      pltpu.SMEM((48,), jnp.int32),
      pltpu.SemaphoreType.DMA, pltpu.SemaphoreType.DMA,
  ]

  @functools.partial(pl.kernel, out_type=out_type, mesh=MESH,
                     scratch_types=scratch, compiler_params=SC_PARAMS)
  def conv1(rsk, rf0, rf1, rdst, boff, x_h, a16, dg,
            xloc, acc, vdeg, k0, k1, a0, a1b, c0, c1b, d0, d1,
            vboff, spm, smb, sr0, sr1):
    w = _wid()
    iota = IOTA()
    zf = jnp.zeros((16,), jnp.float32)

    pltpu.sync_copy(x_h, xloc)
    pltpu.sync_copy(boff, vboff)
    pltpu.sync_copy(boff, spm)
    pltpu.sync_copy(spm, smb)

    def zrow(r, carry):
      acc[pl.ds(r * 16, 16)] = zf
      return carry
    lax.fori_loop(0, BS, zrow, 0)

    def zdeg(r, carry):
      vdeg[pl.ds(r * 16, 16)] = zf
      return carry
    lax.fori_loop(0, BS // 16, zdeg, 0)

    start = smb[w]
    end = smb[w + 1]
    p0s = start & (~15)
    n = end - p0s
    T = lax.div(n + 2 * C - 1, 2 * C)
    vstart = plsc.load_gather(vboff, [_vfull_i(w)])
    vend = plsc.load_gather(vboff, [_vfull_i(w + 1)])
    vbs = _vfull_i(w * BS)
    jmax = jnp.maximum(2 * T - 1, 0)

    def fire_rec(j, ks, af0, af1, dd, sem):
      jc = jnp.clip(j, 0, jmax)
      p = pl.multiple_of(p0s + jc * C, 16)
      pltpu.make_async_copy(rsk.at[pl.ds(p, C)], ks, sem).start()
      pltpu.make_async_copy(rf0.at[pl.ds(p, C)], af0, sem).start()
      pltpu.make_async_copy(rf1.at[pl.ds(p, C)], af1, sem).start()
      pltpu.make_async_copy(rdst.at[pl.ds(p, C)], dd, sem).start()

    def wait_rec(ks, af0, af1, dd, sem):
      pltpu.make_async_copy(rsk.at[pl.ds(0, C)], ks, sem).wait()
      pltpu.make_async_copy(rf0.at[pl.ds(0, C)], af0, sem).wait()
      pltpu.make_async_copy(rf1.at[pl.ds(0, C)], af1, sem).wait()
      pltpu.make_async_copy(rdst.at[pl.ds(0, C)], dd, sem).wait()

    sets = ((k0, a0, c0, d0, sr0), (k1, a1b, c1b, d1, sr1))
    fire_rec(0, k0, a0, c0, d0, sr0)
    fire_rec(1, k1, a1b, c1b, d1, sr1)

    def pair(t, carry):
      for jj, (ks, af0, af1, dd, sem) in enumerate(sets):
        j = 2 * t + jj
        wait_rec(ks, af0, af1, dd, sem)
        pb = p0s + j * C
        for g in range(C // 16):
          off = g * 16
          sk = ks[pl.ds(off, 16)]
          f0 = af0[pl.ds(off, 16)]
          f1 = af1[pl.ds(off, 16)]
          d16 = dd[pl.ds(off, 16)]
          ev = _vfull_i(pb) + (off + iota)
          valid = (ev >= vstart) & (ev < vend)
          sk = jnp.where(valid, sk, 0)
          src = sk & (N - 1)
          i0 = (sk >> SHN) & 3
          i1 = sk >> (SHN + 2)
          k00 = (i0 << 2) + i1
          xs = plsc.load_gather(xloc, [src])
          omf0 = 1.0 - f0
          omf1 = 1.0 - f1
          w00 = jnp.where(valid, omf0 * omf1, zf)
          w01 = jnp.where(valid, omf0 * f1, zf)
          w10 = jnp.where(valid, f0 * omf1, zf)
          w11 = jnp.where(valid, f0 * f1, zf)
          dloc = jnp.clip(d16 - vbs, 0, BS - 1)
          dbase = (dloc << 4) + k00
          for dk, wt in ((0, w00), (1, w01), (4, w10), (5, w11)):
            plsc.addupdate_scatter(acc, [dbase + dk], wt * xs)
          plsc.addupdate_scatter(vdeg, [dloc], jnp.where(valid, 1.0, 0.0))
        fire_rec(j + 2, ks, af0, af1, dd, sem)
      return carry

    lax.fori_loop(0, T, pair, 0)
    for (ks, af0, af1, dd, sem) in sets:
      wait_rec(ks, af0, af1, dd, sem)
    pltpu.sync_copy(acc, a16.at[pl.ds(w * (BS * 16), BS * 16)])
    pltpu.sync_copy(vdeg, dg.at[pl.ds(w * BS, BS)])

  return conv1


def _make_conv_t(N, E, SHB, SHN, cch, C):
  BS = N // NW
  EP = E + PAD

  out_type = jax.ShapeDtypeStruct((N * cch,), jnp.float32)
  scratch = [
      pltpu.VMEM((BS * cch,), jnp.float32),
      pltpu.VMEM((C,), jnp.int32), pltpu.VMEM((C,), jnp.int32),
      pltpu.VMEM((C,), jnp.float32), pltpu.VMEM((C,), jnp.float32),
      pltpu.VMEM((C,), jnp.float32), pltpu.VMEM((C,), jnp.float32),
      pltpu.VMEM((C,), jnp.int32), pltpu.VMEM((C,), jnp.int32),
      pltpu.VMEM((C,), jnp.int32), pltpu.VMEM((C,), jnp.int32),
      pltpu.VMEM((C,), jnp.int32), pltpu.VMEM((C,), jnp.int32),
      pltpu.VMEM((C, 4 * cch), jnp.float32), pltpu.VMEM((C, 4 * cch), jnp.float32),
      pltpu.VMEM((C, 4 * cch), jnp.float32), pltpu.VMEM((C, 4 * cch), jnp.float32),
      pltpu.VMEM((48,), jnp.int32),
      pltpu.VMEM_SHARED((48,), jnp.int32),
      pltpu.SMEM((48,), jnp.int32),
      pltpu.SemaphoreType.DMA, pltpu.SemaphoreType.DMA,
      pltpu.SemaphoreType.DMA, pltpu.SemaphoreType.DMA,
  ]

  @functools.partial(pl.kernel, out_type=out_type, mesh=MESH,
                     scratch_types=scratch, compiler_params=SC_PARAMS)
  def conv_t(rsk, rf0, rf1, rdst, boff, table, s_out,
             acc, k0, k1, a0, a1b, c0, c1b, d0, d1,
             ia0, ia1, ib0, ib1, ra0, ra1, rb0, rb1,
             vboff, spm, smb, sr0, sr1, sg0, sg1):
    w = _wid()
    iota = IOTA()
    zf = jnp.zeros((16,), jnp.float32)

    pltpu.sync_copy(boff, vboff)
    pltpu.sync_copy(boff, spm)
    pltpu.sync_copy(spm, smb)

    def zrow(r, carry):
      acc[pl.ds(r * 16, 16)] = zf
      return carry
    lax.fori_loop(0, BS * cch // 16, zrow, 0)

    start = smb[w]
    end = smb[w + 1]
    p0s = start & (~15)
    n = end - p0s
    T = lax.div(n + 2 * C - 1, 2 * C)
    vstart = plsc.load_gather(vboff, [_vfull_i(w)])
    vend = plsc.load_gather(vboff, [_vfull_i(w + 1)])
    vbs = _vfull_i(w * BS)
    jmax = jnp.maximum(2 * T, 0)

    def fire_rec(j, ks, af0, af1, dd, sem):
      jc = jnp.clip(j, 0, jmax)
      p = pl.multiple_of(p0s + jc * C, 16)
      pltpu.make_async_copy(rsk.at[pl.ds(p, C)], ks, sem).start()
      pltpu.make_async_copy(rf0.at[pl.ds(p, C)], af0, sem).start()
      pltpu.make_async_copy(rf1.at[pl.ds(p, C)], af1, sem).start()
      pltpu.make_async_copy(rdst.at[pl.ds(p, C)], dd, sem).start()

    def wait_rec(ks, af0, af1, dd, sem):
      pltpu.make_async_copy(rsk.at[pl.ds(0, C)], ks, sem).wait()
      pltpu.make_async_copy(rf0.at[pl.ds(0, C)], af0, sem).wait()
      pltpu.make_async_copy(rf1.at[pl.ds(0, C)], af1, sem).wait()
      pltpu.make_async_copy(rdst.at[pl.ds(0, C)], dd, sem).wait()

    def valid_of(j, off):
      ev = _vfull_i(p0s + j * C) + (off + iota)
      return (ev >= vstart) & (ev < vend)

    def build(j, ks, ia, ib, sg, ra, rb):
      for g in range(C // 16):
        off = g * 16
        sk = jnp.where(valid_of(j, off), ks[pl.ds(off, 16)], 0)
        rowa = sk & (4 * N - 1)
        rowb = jnp.minimum(rowa + N, 4 * N - 1)
        ia[pl.ds(off, 16)] = rowa
        ib[pl.ds(off, 16)] = rowb
      pltpu.make_async_copy(table.at[ia], ra, sg).start()
      pltpu.make_async_copy(table.at[ib], rb, sg).start()

    def wait_g(table_ref, ia, ra, rb, sg):
      pltpu.make_async_copy(table_ref.at[ia], ra, sg).wait()
      pltpu.make_async_copy(table_ref.at[ia], rb, sg).wait()

    def compute(j, ks, af0, af1, dd, ra, rb):
      for g in range(C // 16):
        off = g * 16
        e16 = off + iota
        valid = valid_of(j, off)
        sk = jnp.where(valid, ks[pl.ds(off, 16)], 0)
        f0 = af0[pl.ds(off, 16)]
        f1 = af1[pl.ds(off, 16)]
        d16 = dd[pl.ds(off, 16)]
        i1 = sk >> (SHN + 2)
        omf0 = 1.0 - f0
        omf1 = 1.0 - f1
        w00 = jnp.where(valid, omf0 * omf1, zf)
        w01 = jnp.where(valid, omf0 * f1, zf)
        w10 = jnp.where(valid, f0 * omf1, zf)
        w11 = jnp.where(valid, f0 * f1, zf)
        dloc = jnp.clip(d16 - vbs, 0, BS - 1)
        idx0 = i1 * cch
        idx1 = idx0 + cch
        iacc = dloc * cch
        for ch in range(cch):
          g00 = plsc.load_gather(ra, [e16, idx0])
          g01 = plsc.load_gather(ra, [e16, idx1])
          g10 = plsc.load_gather(rb, [e16, idx0])
          g11 = plsc.load_gather(rb, [e16, idx1])
          ms = w00 * g00 + w01 * g01 + w10 * g10 + w11 * g11
          plsc.addupdate_scatter(acc, [iacc], ms)
          idx0 = idx0 + 1
          idx1 = idx1 + 1
          iacc = iacc + 1

    sets = (
        (k0, a0, c0, d0, ia0, ib0, ra0, rb0, sr0, sg0),
        (k1, a1b, c1b, d1, ia1, ib1, ra1, rb1, sr1, sg1),
    )
    fire_rec(0, k0, a0, c0, d0, sr0)
    wait_rec(k0, a0, c0, d0, sr0)
    build(0, k0, ia0, ib0, sg0, ra0, rb0)
    fire_rec(1, k1, a1b, c1b, d1, sr1)

    def pair(t, carry):
      (k_0, a_0, c_0, d_0, iaa0, ibb0, raa0, rbb0, srr0, sgg0) = sets[0]
      (k_1, a_1, c_1, d_1, iaa1, ibb1, raa1, rbb1, srr1, sgg1) = sets[1]
      j0 = 2 * t
      j1 = 2 * t + 1
      wait_rec(k_1, a_1, c_1, d_1, srr1)
      build(j1, k_1, iaa1, ibb1, sgg1, raa1, rbb1)
      wait_g(table, iaa0, raa0, rbb0, sgg0)
      compute(j0, k_0, a_0, c_0, d_0, raa0, rbb0)
      fire_rec(j0 + 2, k_0, a_0, c_0, d_0, srr0)
      wait_rec(k_0, a_0, c_0, d_0, srr0)
      build(j0 + 2, k_0, iaa0, ibb0, sgg0, raa0, rbb0)
      wait_g(table, iaa1, raa1, rbb1, sgg1)
      compute(j1, k_1, a_1, c_1, d_1, raa1, rbb1)
      fire_rec(j1 + 2, k_1, a_1, c_1, d_1, srr1)
      return carry

    lax.fori_loop(0, T, pair, 0)
    wait_g(table, ia0, ra0, rb0, sg0)
    wait_rec(k1, a1b, c1b, d1, sr1)
    pltpu.sync_copy(acc, s_out.at[pl.ds(w * (BS * cch), BS * cch)])

  return conv_t


def _elu(h):
  return jnp.where(h > 0, h, jnp.exp(h) - 1.0)


def _t1_call(a16, x2d, dg, w1f, r1, b1, w2f):
  B = 512

  def fn(a_r, x_r, d_r, w1_r, r1_r, b1_r, w2_r, h1_o, t2_o):
    a = jnp.dot(a_r[...], w1_r[...], preferred_element_type=jnp.float32)
    d = jnp.maximum(d_r[...], 1.0)
    h = a / d + x_r[...] * r1_r[...] + b1_r[...]
    h = _elu(h)
    h1_o[...] = h
    t = jnp.dot(h, w2_r[...], preferred_element_type=jnp.float32)
    t2_o[...] = t.reshape(B, 4, 128).transpose(1, 0, 2)

  return pl.pallas_call(
      fn,
      grid=(N0 // B,),
      in_specs=[
          pl.BlockSpec((B, 16), lambda i: (i, 0)),
          pl.BlockSpec((B, 1), lambda i: (i, 0)),
          pl.BlockSpec((B, 1), lambda i: (i, 0)),
          pl.BlockSpec((16, 32), lambda i: (0, 0)),
          pl.BlockSpec((1, 32), lambda i: (0, 0)),
          pl.BlockSpec((1, 32), lambda i: (0, 0)),
          pl.BlockSpec((32, 512), lambda i: (0, 0)),
      ],
      out_specs=[
          pl.BlockSpec((B, 32), lambda i: (i, 0)),
          pl.BlockSpec((4, B, 128), lambda i: (0, i, 0)),
      ],
      out_shape=[
          jax.ShapeDtypeStruct((N0, 32), jnp.float32),
          jax.ShapeDtypeStruct((4, N0, 128), jnp.float32),
      ],
  )(a16, x2d, dg, w1f, r1, b1, w2f)


def _t2_call(s2, h1, dg, r2, b2, w3f):
  B = 512

  def fn(s_r, h_r, d_r, r2_r, b2_r, w3_r, hp_o, t3_o):
    d = jnp.maximum(d_r[...], 1.0)
    h = s_r[...] / d + jnp.dot(h_r[...], r2_r[...],
                               preferred_element_type=jnp.float32) + b2_r[...]
    h = _elu(h)
    hp = h.reshape(B // 2, 2, 32).max(axis=1)
    hp_o[...] = hp
    t = jnp.dot(hp, w3_r[...], preferred_element_type=jnp.float32)
    t3_o[...] = t.reshape(B // 2, 4, 256).transpose(1, 0, 2)

  return pl.pallas_call(
      fn,
      grid=(N0 // B,),
      in_specs=[
          pl.BlockSpec((B, 32), lambda i: (i, 0)),
          pl.BlockSpec((B, 32), lambda i: (i, 0)),
          pl.BlockSpec((B, 1), lambda i: (i, 0)),
          pl.BlockSpec((32, 32), lambda i: (0, 0)),
          pl.BlockSpec((1, 32), lambda i: (0, 0)),
          pl.BlockSpec((32, 1024), lambda i: (0, 0)),
      ],
      out_specs=[
          pl.BlockSpec((B // 2, 32), lambda i: (i, 0)),
          pl.BlockSpec((4, B // 2, 256), lambda i: (0, i, 0)),
      ],
      out_shape=[
          jax.ShapeDtypeStruct((N1, 32), jnp.float32),
          jax.ShapeDtypeStruct((4, N1, 256), jnp.float32),
      ],
  )(s2, h1, dg, r2, b2, w3f)


def _t3_call(s3, h2p, dg1, r3, b3, w4f):
  B = 512

  def fn(s_r, h_r, d_r, r3_r, b3_r, w4_r, h3_o, t4_o):
    d = jnp.maximum(d_r[...], 1.0)
    h = s_r[...] / d + jnp.dot(h_r[...], r3_r[...],
                               preferred_element_type=jnp.float32) + b3_r[...]
    h = _elu(h)
    h3_o[...] = h
    t = jnp.dot(h, w4_r[...], preferred_element_type=jnp.float32)
    t4_o[...] = t.reshape(B, 4, 256).transpose(1, 0, 2)

  return pl.pallas_call(
      fn,
      grid=(N1 // B,),
      in_specs=[
          pl.BlockSpec((B, 64), lambda i: (i, 0)),
          pl.BlockSpec((B, 32), lambda i: (i, 0)),
          pl.BlockSpec((B, 1), lambda i: (i, 0)),
          pl.BlockSpec((32, 64), lambda i: (0, 0)),
          pl.BlockSpec((1, 64), lambda i: (0, 0)),
          pl.BlockSpec((64, 1024), lambda i: (0, 0)),
      ],
      out_specs=[
          pl.BlockSpec((B, 64), lambda i: (i, 0)),
          pl.BlockSpec((4, B, 256), lambda i: (0, i, 0)),
      ],
      out_shape=[
          jax.ShapeDtypeStruct((N1, 64), jnp.float32),
          jax.ShapeDtypeStruct((4, N1, 256), jnp.float32),
      ],
  )(s3, h2p, dg1, r3, b3, w4f)


def _t4_call(s4, h3, dg1, r4, b4):
  B = 512

  def fn(s_r, h_r, d_r, r4_r, b4_r, hp_o):
    d = jnp.maximum(d_r[...], 1.0)
    h = s_r[...] / d + jnp.dot(h_r[...], r4_r[...],
                               preferred_element_type=jnp.float32) + b4_r[...]
    h = _elu(h)
    hp_o[...] = h.reshape(B // 2, 2, 64).max(axis=1)

  return pl.pallas_call(
      fn,
      grid=(N1 // B,),
      in_specs=[
          pl.BlockSpec((B, 64), lambda i: (i, 0)),
          pl.BlockSpec((B, 64), lambda i: (i, 0)),
          pl.BlockSpec((B, 1), lambda i: (i, 0)),
          pl.BlockSpec((64, 64), lambda i: (0, 0)),
          pl.BlockSpec((1, 64), lambda i: (0, 0)),
      ],
      out_specs=pl.BlockSpec((B // 2, 64), lambda i: (i, 0)),
      out_shape=jax.ShapeDtypeStruct((N1 // 2, 64), jnp.float32),
  )(s4, h3, dg1, r4, b4)


def _t5_call(xfc, fc1_w, fc1_b):
  KB = 64
  BK = 65536 // KB

  def fn(x_r, w_r, b_r, o_r, acc_r):
    k = pl.program_id(0)

    @pl.when(k == 0)
    def _():
      acc_r[...] = jnp.zeros_like(acc_r)

    acc_r[...] += jnp.dot(x_r[...], w_r[...],
                          preferred_element_type=jnp.float32)

    @pl.when(k == KB - 1)
    def _():
      o_r[...] = _elu(acc_r[...] + b_r[...])

  return pl.pallas_call(
      fn,
      grid=(KB,),
      in_specs=[
          pl.BlockSpec((16, BK), lambda k: (0, k)),
          pl.BlockSpec((BK, 512), lambda k: (k, 0)),
          pl.BlockSpec((1, 512), lambda k: (0, 0)),
      ],
      out_specs=pl.BlockSpec((16, 512), lambda k: (0, 0)),
      out_shape=jax.ShapeDtypeStruct((16, 512), jnp.float32),
      scratch_shapes=[pltpu.VMEM((16, 512), jnp.float32)],
  )(xfc, fc1_w, fc1_b)


def _t6_call(h5, fc2_w, fc2_b):
  def fn(h_r, w_r, b_r, o_r):
    z = jnp.dot(h_r[...], w_r[...],
                preferred_element_type=jnp.float32) + b_r[...]
    m = jnp.max(z, axis=-1, keepdims=True)
    e = jnp.exp(z - m)
    s = jnp.sum(e, axis=-1, keepdims=True)
    o_r[...] = z - m - jnp.log(s)

  return pl.pallas_call(
      fn,
      out_shape=jax.ShapeDtypeStruct((16, 10), jnp.float32),
  )(h5, fc2_w, fc2_b)


_a1_l0 = _make_a1(E0, 11)
_a2_l0 = _make_a2(N0, E0, 11, 16)
_conv1_l0 = _make_conv1(N0, E0, 11, 16)
_conv2_l0 = _make_conv_t(N0, E0, 11, 16, 32, 64)
_a1_l1 = _make_a1(E1, 10)
_a2_l1 = _make_a2(N1, E1, 10, 15)
_conv1_l1 = _make_conv1(N1, E1, 10, 15)
_conv3_l1 = _make_conv_t(N1, E1, 10, 15, 64, 32)
_conv4_l1 = _make_conv_t(N1, E1, 10, 15, 64, 32)


def kernel(x, edge_index0, pseudo0, edge_index1, pseudo1,
           W1, root1, b1, W2, root2, b2, W3, root3, b3, W4, root4, b4,
           fc1_w, fc1_b, fc2_w, fc2_b):
  src0 = edge_index0[0]
  dst0 = edge_index0[1]
  ps0 = pseudo0.reshape(-1)
  src1 = edge_index1[0]
  dst1 = edge_index1[1]
  ps1 = pseudo1.reshape(-1)

  hist0 = _a1_l0(dst0)
  rsk0, rf00, rf10, rdst0, boff0 = _a2_l0(src0, dst0, ps0, hist0)
  a16, dg0 = _conv1_l0(rsk0, rf00, rf10, rdst0, boff0, x[:, 0])
  a16 = a16.reshape(N0, 16)
  dg0c = dg0.reshape(N0, 1)

  w1f = W1.reshape(16, 32)
  w2f = W2.transpose(1, 0, 2).reshape(32, 512)
  h1, t2 = _t1_call(a16, x, dg0c, w1f, root1.reshape(1, 32),
                    b1.reshape(1, 32), w2f)
  s2 = _conv2_l0(rsk0, rf00, rf10, rdst0, boff0,
                 t2.reshape(4 * N0, 128)).reshape(N0, 32)

  w3f = W3.transpose(1, 0, 2).reshape(32, 1024)
  h2p, t3 = _t2_call(s2, h1, dg0c, root2, b2.reshape(1, 32), w3f)

  hist1 = _a1_l1(dst1)
  rsk1, rf01, rf11, rdst1, boff1 = _a2_l1(src1, dst1, ps1, hist1)
  _, dg1 = _conv1_l1(rsk1, rf01, rf11, rdst1, boff1,
                     jnp.zeros((N1,), jnp.float32))
  dg1c = dg1.reshape(N1, 1)

  s3 = _conv3_l1(rsk1, rf01, rf11, rdst1, boff1,
                 t3.reshape(4 * N1, 256)).reshape(N1, 64)
  w4f = W4.transpose(1, 0, 2).reshape(64, 1024)
  h3, t4 = _t3_call(s3, h2p, dg1c, root3, b3.reshape(1, 64), w4f)
  s4 = _conv4_l1(rsk1, rf01, rf11, rdst1, boff1,
                 t4.reshape(4 * N1, 256)).reshape(N1, 64)
  h4p = _t4_call(s4, h3, dg1c, root4, b4.reshape(1, 64))

  xfc = h4p.reshape(16, 65536)
  h5 = _t5_call(xfc, fc1_w, fc1_b.reshape(1, 512))
  return _t6_call(h5, fc2_w, fc2_b.reshape(1, 10))


# speedup vs baseline: 1.9888x; 1.0145x over previous
import functools
import jax
import jax.numpy as jnp
from jax import lax
from jax.experimental import pallas as pl
from jax.experimental.pallas import tpu as pltpu
from jax.experimental.pallas import tpu_sc as plsc

NW = 32
N0, E0 = 65536, 1048576
N1, E1 = 32768, 524288
PAD = 1024

MESH = plsc.VectorSubcoreMesh(core_axis_name="c", subcore_axis_name="s")
SC_PARAMS = pltpu.CompilerParams(needs_layout_passes=False)
IOTA = lambda: lax.iota(jnp.int32, 16)


def _wid():
  return lax.axis_index("s") * 2 + lax.axis_index("c")


def _vfull_i(v):
  return jnp.full((16,), 1, jnp.int32) * v


def _vfull_f(v):
  return jnp.full((16,), 1.0, jnp.float32) * v


def _make_a1(E, SHB):
  C = 512
  NCH = E // NW // C

  @functools.partial(
      pl.kernel,
      out_type=jax.ShapeDtypeStruct((NW, 32), jnp.int32),
      mesh=MESH,
      scratch_types=[
          pltpu.VMEM((C,), jnp.int32), pltpu.VMEM((C,), jnp.int32),
          pltpu.VMEM((32,), jnp.int32),
          pltpu.SemaphoreType.DMA, pltpu.SemaphoreType.DMA,
      ],
      compiler_params=SC_PARAMS,
  )
  def a1(dst_h, hist, b0, b1, vh, sr0, sr1):
    w = _wid()
    base = w * (E // NW)
    zero = jnp.zeros((16,), jnp.int32)
    vh[pl.ds(0, 16)] = zero
    vh[pl.ds(16, 16)] = zero
    ones = jnp.full((16,), 1, jnp.int32)

    def fire(j, buf, sem):
      jc = jnp.minimum(j, NCH - 1)
      p = pl.multiple_of(base + jc * C, 256)
      pltpu.make_async_copy(dst_h.at[pl.ds(p, C)], buf, sem).start()

    def wait(buf, sem):
      pltpu.make_async_copy(dst_h.at[pl.ds(base, C)], buf, sem).wait()

    fire(0, b0, sr0)
    fire(1, b1, sr1)

    def pair(t, carry):
      for jj, (buf, sem) in enumerate(((b0, sr0), (b1, sr1))):
        j = 2 * t + jj
        wait(buf, sem)
        for g in range(C // 16):
          d16 = buf[pl.ds(g * 16, 16)]
          bk = d16 >> SHB
          plsc.addupdate_scatter(vh, [bk], ones)
        fire(j + 2, buf, sem)
      return carry

    lax.fori_loop(0, NCH // 2, pair, 0)
    wait(b0, sr0)
    wait(b1, sr1)
    pltpu.sync_copy(vh, hist.at[w])

  return a1


def _make_a2(N, E, SHB, SHN):
  C = 256
  NCH = E // NW // C
  EP = E + PAD

  out_type = (
      jax.ShapeDtypeStruct((EP,), jnp.int32),
      jax.ShapeDtypeStruct((EP,), jnp.float32),
      jax.ShapeDtypeStruct((EP,), jnp.float32),
      jax.ShapeDtypeStruct((EP,), jnp.int32),
      jax.ShapeDtypeStruct((48,), jnp.int32),
  )
  scratch = [
      pltpu.VMEM((C,), jnp.int32), pltpu.VMEM((C,), jnp.int32),
      pltpu.VMEM((C,), jnp.int32), pltpu.VMEM((C,), jnp.int32),
      pltpu.VMEM((2 * C,), jnp.float32), pltpu.VMEM((2 * C,), jnp.float32),
      pltpu.VMEM((NW, 32), jnp.int32),
      pltpu.VMEM((32,), jnp.int32),
      pltpu.VMEM((32,), jnp.int32),
      pltpu.VMEM((16,), jnp.int32),
      pltpu.VMEM((C,), jnp.int32), pltpu.VMEM((C,), jnp.int32),
      pltpu.VMEM((C,), jnp.float32), pltpu.VMEM((C,), jnp.float32),
      pltpu.VMEM((C,), jnp.float32), pltpu.VMEM((C,), jnp.float32),
      pltpu.VMEM((C,), jnp.int32), pltpu.VMEM((C,), jnp.int32),
      pltpu.VMEM((48,), jnp.int32),
      pltpu.SemaphoreType.DMA, pltpu.SemaphoreType.DMA,
      pltpu.SemaphoreType.DMA, pltpu.SemaphoreType.DMA,
  ]

  @functools.partial(pl.kernel, out_type=out_type, mesh=MESH,
                     scratch_types=scratch, compiler_params=SC_PARAMS)
  def a2(src_h, dst_h, ps_h, hist, rsk, rf0, rf1, rdst, boff,
         bs0, bs1, bd0, bd1, bp0, bp1, vhist, vbase, vpre, vbk,
         tsk0, tsk1, tf00, tf01, tf10, tf11, td0, td1, tboff,
         sr0, sr1, ss0, ss1):
    w = _wid()
    ew = E // NW
    base = w * ew
    iota = IOTA()
    ones = jnp.full((16,), 1, jnp.int32)

    pltpu.sync_copy(hist, vhist)
    wv = _vfull_i(w)
    acc0 = jnp.zeros((16,), jnp.int32)
    acc1 = jnp.zeros((16,), jnp.int32)
    tot0 = jnp.zeros((16,), jnp.int32)
    tot1 = jnp.zeros((16,), jnp.int32)
    for wp in range(NW):
      r0 = vhist[wp, pl.ds(0, 16)]
      r1 = vhist[wp, pl.ds(16, 16)]
      m = jnp.where(jnp.full((16,), wp, jnp.int32) < wv, 1, 0)
      acc0 = acc0 + r0 * m
      acc1 = acc1 + r1 * m
      tot0 = tot0 + r0
      tot1 = tot1 + r1
    vpre[pl.ds(0, 16)] = tot0
    vpre[pl.ds(16, 16)] = tot1
    g0 = iota
    g1 = iota + 16
    for s in (1, 2, 4, 8, 16):
      a0 = vpre[pl.ds(0, 16)]
      a1v = vpre[pl.ds(16, 16)]
      s0 = plsc.load_gather(vpre, [jnp.maximum(g0 - s, 0)])
      s1 = plsc.load_gather(vpre, [jnp.maximum(g1 - s, 0)])
      a0 = a0 + jnp.where(g0 >= s, s0, 0)
      a1v = a1v + jnp.where(g1 >= s, s1, 0)
      vpre[pl.ds(0, 16)] = a0
      vpre[pl.ds(16, 16)] = a1v
    e0 = jnp.where(g0 >= 1, plsc.load_gather(vpre, [jnp.maximum(g0 - 1, 0)]), 0)
    e1 = plsc.load_gather(vpre, [g1 - 1])
    vbase[pl.ds(0, 16)] = e0 + acc0
    vbase[pl.ds(16, 16)] = e1 + acc1

    @pl.when(w == 0)
    def _():
      tboff[pl.ds(0, 16)] = e0
      tboff[pl.ds(16, 16)] = e1
      tboff[pl.ds(32, 16)] = jnp.full((16,), E, jnp.int32)
      pltpu.sync_copy(tboff, boff)

    def fire_rec(j, bs, bd, bp, sem):
      jc = jnp.minimum(j, NCH - 1)
      p = pl.multiple_of(base + jc * C, 256)
      pltpu.make_async_copy(src_h.at[pl.ds(p, C)], bs, sem).start()
      pltpu.make_async_copy(dst_h.at[pl.ds(p, C)], bd, sem).start()
      p2 = pl.multiple_of(2 * p, 512)
      pltpu.make_async_copy(ps_h.at[pl.ds(p2, 2 * C)], bp, sem).start()

    def wait_rec(bs, bd, bp, sem):
      pltpu.make_async_copy(src_h.at[pl.ds(base, C)], bs, sem).wait()
      pltpu.make_async_copy(dst_h.at[pl.ds(base, C)], bd, sem).wait()
      pltpu.make_async_copy(ps_h.at[pl.ds(base, 2 * C)], bp, sem).wait()

    def drain_scat(tsk, tf0, tf1, td, sem):
      pltpu.make_async_copy(rsk.at[pl.ds(0, C)], tsk, sem).wait()
      pltpu.make_async_copy(rf0.at[pl.ds(0, C)], tf0, sem).wait()
      pltpu.make_async_copy(rf1.at[pl.ds(0, C)], tf1, sem).wait()
      pltpu.make_async_copy(rdst.at[pl.ds(0, C)], td, sem).wait()

    sets = (
        (bs0, bd0, bp0, sr0, tsk0, tf00, tf10, td0, ss0),
        (bs1, bd1, bp1, sr1, tsk1, tf01, tf11, td1, ss1),
    )
    fire_rec(0, bs0, bd0, bp0, sr0)
    fire_rec(1, bs1, bd1, bp1, sr1)

    def pair(t, carry):
      for jj, (bs, bd, bp, srx, tsk, tf0, tf1, td, ssx) in enumerate(sets):
        j = 2 * t + jj

        @pl.when(j >= 2)
        def _():
          drain_scat(tsk, tf0, tf1, td, ssx)

        wait_rec(bs, bd, bp, srx)
        for g in range(C // 16):
          off = g * 16
          s16 = bs[pl.ds(off, 16)]
          d16 = bd[pl.ds(off, 16)]
          p0 = plsc.load_gather(bp, [(off + iota) * 2])
          p1 = plsc.load_gather(bp, [(off + iota) * 2 + 1])
          pos0 = p0 * 3.0
          pos1 = p1 * 3.0
          i0 = jnp.clip(pos0.astype(jnp.int32), 0, 2)
          i1 = jnp.clip(pos1.astype(jnp.int32), 0, 2)
          f0 = pos0 - i0.astype(jnp.float32)
          f1 = pos1 - i1.astype(jnp.float32)
          sk = s16 | (i0 << SHN) | (i1 << (SHN + 2))
          bk = d16 >> SHB
          vbk[...] = bk
          rank = jnp.zeros((16,), jnp.int32)
          for sft in range(1, 16):
            other = plsc.load_gather(vbk, [(iota - sft) & 15])
            hit = (other == bk) & (iota >= sft)
            rank = rank + jnp.where(hit, 1, 0)
          bb = plsc.load_gather(vbase, [bk])
          pos = bb + rank
          plsc.addupdate_scatter(vbase, [bk], ones)
          tsk[pl.ds(off, 16)] = sk
          tf0[pl.ds(off, 16)] = f0
          tf1[pl.ds(off, 16)] = f1
          td[pl.ds(off, 16)] = d16
          pltpu.make_async_copy(tsk.at[pl.ds(off, 16)], rsk.at[pos], ssx).start()
          pltpu.make_async_copy(tf0.at[pl.ds(off, 16)], rf0.at[pos], ssx).start()
          pltpu.make_async_copy(tf1.at[pl.ds(off, 16)], rf1.at[pos], ssx).start()
          pltpu.make_async_copy(td.at[pl.ds(off, 16)], rdst.at[pos], ssx).start()
        fire_rec(j + 2, bs, bd, bp, srx)
      return carry

    lax.fori_loop(0, NCH // 2, pair, 0)
    for (bs, bd, bp, srx, tsk, tf0, tf1, td, ssx) in sets:
      wait_rec(bs, bd, bp, srx)
      drain_scat(tsk, tf0, tf1, td, ssx)

  return a2


def _make_conv1(N, E, SHB, SHN):
  C = 256
  BS = N // NW
  EP = E + PAD

  out_type = (
      jax.ShapeDtypeStruct((N * 16,), jnp.float32),
      jax.ShapeDtypeStruct((N,), jnp.float32),
  )
  scratch = [
      pltpu.VMEM((N,), jnp.float32),
      pltpu.VMEM((BS * 16,), jnp.float32),
      pltpu.VMEM((BS,), jnp.float32),
      pltpu.VMEM((C,), jnp.int32), pltpu.VMEM((C,), jnp.int32),
      pltpu.VMEM((C,), jnp.float32), pltpu.VMEM((C,), jnp.float32),
      pltpu.VMEM((C,), jnp.float32), pltpu.VMEM((C,), jnp.float32),
      pltpu.VMEM((C,), jnp.int32), pltpu.VMEM((C,), jnp.int32),
      pltpu.VMEM((48,), jnp.int32),
      pltpu.VMEM_SHARED((48,), jnp.int32),
      pltpu.SMEM((48,), jnp.int32),
      pltpu.SemaphoreType.DMA, pltpu.SemaphoreType.DMA,
  ]

  @functools.partial(pl.kernel, out_type=out_type, mesh=MESH,
                     scratch_types=scratch, compiler_params=SC_PARAMS)
  def conv1(rsk, rf0, rf1, rdst, boff, x_h, a16, dg,
            xloc, acc, vdeg, k0, k1, a0, a1b, c0, c1b, d0, d1,
            vboff, spm, smb, sr0, sr1):
    w = _wid()
    iota = IOTA()
    zf = jnp.zeros((16,), jnp.float32)

    pltpu.sync_copy(x_h, xloc)
    pltpu.sync_copy(boff, vboff)
    pltpu.sync_copy(boff, spm)
    pltpu.sync_copy(spm, smb)

    def zrow(r, carry):
      acc[pl.ds(r * 16, 16)] = zf
      return carry
    lax.fori_loop(0, BS, zrow, 0)

    def zdeg(r, carry):
      vdeg[pl.ds(r * 16, 16)] = zf
      return carry
    lax.fori_loop(0, BS // 16, zdeg, 0)

    start = smb[w]
    end = smb[w + 1]
    p0s = start & (~15)
    n = end - p0s
    T = lax.div(n + 2 * C - 1, 2 * C)
    vstart = plsc.load_gather(vboff, [_vfull_i(w)])
    vend = plsc.load_gather(vboff, [_vfull_i(w + 1)])
    vbs = _vfull_i(w * BS)
    jmax = jnp.maximum(2 * T - 1, 0)

    def fire_rec(j, ks, af0, af1, dd, sem):
      jc = jnp.clip(j, 0, jmax)
      p = pl.multiple_of(p0s + jc * C, 16)
      pltpu.make_async_copy(rsk.at[pl.ds(p, C)], ks, sem).start()
      pltpu.make_async_copy(rf0.at[pl.ds(p, C)], af0, sem).start()
      pltpu.make_async_copy(rf1.at[pl.ds(p, C)], af1, sem).start()
      pltpu.make_async_copy(rdst.at[pl.ds(p, C)], dd, sem).start()

    def wait_rec(ks, af0, af1, dd, sem):
      pltpu.make_async_copy(rsk.at[pl.ds(0, C)], ks, sem).wait()
      pltpu.make_async_copy(rf0.at[pl.ds(0, C)], af0, sem).wait()
      pltpu.make_async_copy(rf1.at[pl.ds(0, C)], af1, sem).wait()
      pltpu.make_async_copy(rdst.at[pl.ds(0, C)], dd, sem).wait()

    sets = ((k0, a0, c0, d0, sr0), (k1, a1b, c1b, d1, sr1))
    fire_rec(0, k0, a0, c0, d0, sr0)
    fire_rec(1, k1, a1b, c1b, d1, sr1)

    def pair(t, carry):
      for jj, (ks, af0, af1, dd, sem) in enumerate(sets):
        j = 2 * t + jj
        wait_rec(ks, af0, af1, dd, sem)
        pb = p0s + j * C
        for g in range(C // 16):
          off = g * 16
          sk = ks[pl.ds(off, 16)]
          f0 = af0[pl.ds(off, 16)]
          f1 = af1[pl.ds(off, 16)]
          d16 = dd[pl.ds(off, 16)]
          ev = _vfull_i(pb) + (off + iota)
          valid = (ev >= vstart) & (ev < vend)
          sk = jnp.where(valid, sk, 0)
          src = sk & (N - 1)
          i0 = (sk >> SHN) & 3
          i1 = sk >> (SHN + 2)
          k00 = (i0 << 2) + i1
          xs = plsc.load_gather(xloc, [src])
          omf0 = 1.0 - f0
          omf1 = 1.0 - f1
          w00 = jnp.where(valid, omf0 * omf1, zf)
          w01 = jnp.where(valid, omf0 * f1, zf)
          w10 = jnp.where(valid, f0 * omf1, zf)
          w11 = jnp.where(valid, f0 * f1, zf)
          dloc = jnp.clip(d16 - vbs, 0, BS - 1)
          dbase = (dloc << 4) + k00
          for dk, wt in ((0, w00), (1, w01), (4, w10), (5, w11)):
            plsc.addupdate_scatter(acc, [dbase + dk], wt * xs)
          plsc.addupdate_scatter(vdeg, [dloc], jnp.where(valid, 1.0, 0.0))
        fire_rec(j + 2, ks, af0, af1, dd, sem)
      return carry

    lax.fori_loop(0, T, pair, 0)
    for (ks, af0, af1, dd, sem) in sets:
      wait_rec(ks, af0, af1, dd, sem)
    pltpu.sync_copy(acc, a16.at[pl.ds(w * (BS * 16), BS * 16)])
    pltpu.sync_copy(vdeg, dg.at[pl.ds(w * BS, BS)])

  return conv1


def _make_deg(N, E):
  C = 512
  BS = N // NW
  EP = E + PAD

  out_type = jax.ShapeDtypeStruct((N,), jnp.float32)
  scratch = [
      pltpu.VMEM((BS,), jnp.float32),
      pltpu.VMEM((C,), jnp.int32), pltpu.VMEM((C,), jnp.int32),
      pltpu.VMEM((48,), jnp.int32),
      pltpu.VMEM_SHARED((48,), jnp.int32),
      pltpu.SMEM((48,), jnp.int32),
      pltpu.SemaphoreType.DMA, pltpu.SemaphoreType.DMA,
  ]

  @functools.partial(pl.kernel, out_type=out_type, mesh=MESH,
                     scratch_types=scratch, compiler_params=SC_PARAMS)
  def degk(rdst, boff, dg, vdeg, d0, d1, vboff, spm, smb, sr0, sr1):
    w = _wid()
    iota = IOTA()
    zf = jnp.zeros((16,), jnp.float32)

    pltpu.sync_copy(boff, vboff)
    pltpu.sync_copy(boff, spm)
    pltpu.sync_copy(spm, smb)

    def zdeg(r, carry):
      vdeg[pl.ds(r * 16, 16)] = zf
      return carry
    lax.fori_loop(0, BS // 16, zdeg, 0)

    start = smb[w]
    end = smb[w + 1]
    p0s = start & (~15)
    n = end - p0s
    T = lax.div(n + 2 * C - 1, 2 * C)
    vstart = plsc.load_gather(vboff, [_vfull_i(w)])
    vend = plsc.load_gather(vboff, [_vfull_i(w + 1)])
    vbs = _vfull_i(w * BS)
    jmax = jnp.maximum(2 * T - 1, 0)

    def fire(j, dd, sem):
      jc = jnp.clip(j, 0, jmax)
      p = pl.multiple_of(p0s + jc * C, 16)
      pltpu.make_async_copy(rdst.at[pl.ds(p, C)], dd, sem).start()

    def wait(dd, sem):
      pltpu.make_async_copy(rdst.at[pl.ds(0, C)], dd, sem).wait()

    sets = ((d0, sr0), (d1, sr1))
    fire(0, d0, sr0)
    fire(1, d1, sr1)

    def pair(t, carry):
      for jj, (dd, sem) in enumerate(sets):
        j = 2 * t + jj
        wait(dd, sem)
        pb = p0s + j * C
        for g in range(C // 16):
          off = g * 16
          d16 = dd[pl.ds(off, 16)]
          ev = _vfull_i(pb) + (off + iota)
          valid = (ev >= vstart) & (ev < vend)
          dloc = jnp.clip(d16 - vbs, 0, BS - 1)
          plsc.addupdate_scatter(vdeg, [dloc], jnp.where(valid, 1.0, 0.0))
        fire(j + 2, dd, sem)
      return carry

    lax.fori_loop(0, T, pair, 0)
    for (dd, sem) in sets:
      wait(dd, sem)
    pltpu.sync_copy(vdeg, dg.at[pl.ds(w * BS, BS)])

  return degk


def _make_conv_t(N, E, SHB, SHN, cch, C):
  BS = N // NW
  EP = E + PAD

  out_type = jax.ShapeDtypeStruct((N * cch,), jnp.float32)
  scratch = [
      pltpu.VMEM((BS * cch,), jnp.float32),
      pltpu.VMEM((C,), jnp.int32), pltpu.VMEM((C,), jnp.int32),
      pltpu.VMEM((C,), jnp.float32), pltpu.VMEM((C,), jnp.float32),
      pltpu.VMEM((C,), jnp.float32), pltpu.VMEM((C,), jnp.float32),
      pltpu.VMEM((C,), jnp.int32), pltpu.VMEM((C,), jnp.int32),
      pltpu.VMEM((C,), jnp.int32), pltpu.VMEM((C,), jnp.int32),
      pltpu.VMEM((C, 4 * cch), jnp.float32), pltpu.VMEM((C, 4 * cch), jnp.float32),
      pltpu.VMEM((48,), jnp.int32),
      pltpu.VMEM_SHARED((48,), jnp.int32),
      pltpu.SMEM((48,), jnp.int32),
      pltpu.SemaphoreType.DMA, pltpu.SemaphoreType.DMA,
      pltpu.SemaphoreType.DMA, pltpu.SemaphoreType.DMA,
  ]

  @functools.partial(pl.kernel, out_type=out_type, mesh=MESH,
                     scratch_types=scratch, compiler_params=SC_PARAMS)
  def conv_t(rsk, rf0, rf1, rdst, boff, table, s_out,
             acc, k0, k1, a0, a1b, c0, c1b, d0, d1,
             ia0, ia1, ra0, ra1,
             vboff, spm, smb, sr0, sr1, sg0, sg1):
    w = _wid()
    iota = IOTA()
    zf = jnp.zeros((16,), jnp.float32)

    pltpu.sync_copy(boff, vboff)
    pltpu.sync_copy(boff, spm)
    pltpu.sync_copy(spm, smb)

    def zrow(r, carry):
      acc[pl.ds(r * 16, 16)] = zf
      return carry
    lax.fori_loop(0, BS * cch // 16, zrow, 0)

    start = smb[w]
    end = smb[w + 1]
    p0s = start & (~15)
    n = end - p0s
    T = lax.div(n + 2 * C - 1, 2 * C)
    vstart = plsc.load_gather(vboff, [_vfull_i(w)])
    vend = plsc.load_gather(vboff, [_vfull_i(w + 1)])
    vbs = _vfull_i(w * BS)
    jmax = jnp.maximum(2 * T, 0)

    def fire_rec(j, ks, af0, af1, dd, sem):
      jc = jnp.clip(j, 0, jmax)
      p = pl.multiple_of(p0s + jc * C, 16)
      pltpu.make_async_copy(rsk.at[pl.ds(p, C)], ks, sem).start()
      pltpu.make_async_copy(rf0.at[pl.ds(p, C)], af0, sem).start()
      pltpu.make_async_copy(rf1.at[pl.ds(p, C)], af1, sem).start()
      pltpu.make_async_copy(rdst.at[pl.ds(p, C)], dd, sem).start()

    def wait_rec(ks, af0, af1, dd, sem):
      pltpu.make_async_copy(rsk.at[pl.ds(0, C)], ks, sem).wait()
      pltpu.make_async_copy(rf0.at[pl.ds(0, C)], af0, sem).wait()
      pltpu.make_async_copy(rf1.at[pl.ds(0, C)], af1, sem).wait()
      pltpu.make_async_copy(rdst.at[pl.ds(0, C)], dd, sem).wait()

    def valid_of(j, off):
      ev = _vfull_i(p0s + j * C) + (off + iota)
      return (ev >= vstart) & (ev < vend)

    def build(j, ks, ia, sg, ra):
      for g in range(C // 16):
        off = g * 16
        sk = jnp.where(valid_of(j, off), ks[pl.ds(off, 16)], 0)
        src = sk & (N - 1)
        i0 = (sk >> SHN) & 3
        i1 = sk >> (SHN + 2)
        ia[pl.ds(off, 16)] = (i0 * 3 + i1) * N + src
      pltpu.make_async_copy(table.at[ia], ra, sg).start()

    def wait_g(table_ref, ia, ra, sg):
      pltpu.make_async_copy(table_ref.at[ia], ra, sg).wait()

    def compute(j, ks, af0, af1, dd, ra):
      for g in range(C // 16):
        off = g * 16
        e16 = off + iota
        valid = valid_of(j, off)
        f0 = af0[pl.ds(off, 16)]
        f1 = af1[pl.ds(off, 16)]
        d16 = dd[pl.ds(off, 16)]
        omf0 = 1.0 - f0
        omf1 = 1.0 - f1
        w00 = jnp.where(valid, omf0 * omf1, zf)
        w01 = jnp.where(valid, omf0 * f1, zf)
        w10 = jnp.where(valid, f0 * omf1, zf)
        w11 = jnp.where(valid, f0 * f1, zf)
        dloc = jnp.clip(d16 - vbs, 0, BS - 1)

        def chbody(it, carry):
          cv, ja = carry
          for u in range(4):
            g00 = plsc.load_gather(ra, [e16, cv])
            g01 = plsc.load_gather(ra, [e16, cv + cch])
            g10 = plsc.load_gather(ra, [e16, cv + 2 * cch])
            g11 = plsc.load_gather(ra, [e16, cv + 3 * cch])
            ms = w00 * g00 + w01 * g01 + w10 * g10 + w11 * g11
            plsc.addupdate_scatter(acc, [ja], ms)
            cv = cv + 1
            ja = ja + 1
          return (cv, ja)

        lax.fori_loop(0, cch // 4, chbody,
                      (jnp.zeros((16,), jnp.int32), dloc * cch))

    sets = (
        (k0, a0, c0, d0, ia0, ra0, sr0, sg0),
        (k1, a1b, c1b, d1, ia1, ra1, sr1, sg1),
    )
    fire_rec(0, k0, a0, c0, d0, sr0)
    wait_rec(k0, a0, c0, d0, sr0)
    build(0, k0, ia0, sg0, ra0)
    fire_rec(1, k1, a1b, c1b, d1, sr1)

    def pair(t, carry):
      (k_0, a_0, c_0, d_0, iaa0, raa0, srr0, sgg0) = sets[0]
      (k_1, a_1, c_1, d_1, iaa1, raa1, srr1, sgg1) = sets[1]
      j0 = 2 * t
      j1 = 2 * t + 1
      wait_rec(k_1, a_1, c_1, d_1, srr1)
      build(j1, k_1, iaa1, sgg1, raa1)
      wait_g(table, iaa0, raa0, sgg0)
      compute(j0, k_0, a_0, c_0, d_0, raa0)
      fire_rec(j0 + 2, k_0, a_0, c_0, d_0, srr0)
      wait_rec(k_0, a_0, c_0, d_0, srr0)
      build(j0 + 2, k_0, iaa0, sgg0, raa0)
      wait_g(table, iaa1, raa1, sgg1)
      compute(j1, k_1, a_1, c_1, d_1, raa1)
      fire_rec(j1 + 2, k_1, a_1, c_1, d_1, srr1)
      return carry

    lax.fori_loop(0, T, pair, 0)
    wait_g(table, ia0, ra0, sg0)
    wait_rec(k1, a1b, c1b, d1, sr1)
    pltpu.sync_copy(acc, s_out.at[pl.ds(w * (BS * cch), BS * cch)])

  return conv_t


def _elu(h):
  return jnp.where(h > 0, h, jnp.exp(h) - 1.0)


def _t1_call(a16, x2d, dg, w1f, r1, b1, w2f):
  B = 512

  def fn(a_r, x_r, d_r, w1_r, r1_r, b1_r, w2_r, h1_o, t2_o):
    a = jnp.dot(a_r[...], w1_r[...], preferred_element_type=jnp.float32)
    d = jnp.maximum(d_r[...], 1.0)
    h = a / d + x_r[...] * r1_r[...] + b1_r[...]
    h = _elu(h)
    h1_o[...] = h
    t = jnp.dot(h, w2_r[...], preferred_element_type=jnp.float32)
    t2_o[...] = t.reshape(B, 9, 128).transpose(1, 0, 2)

  return pl.pallas_call(
      fn,
      grid=(N0 // B,),
      in_specs=[
          pl.BlockSpec((B, 16), lambda i: (i, 0)),
          pl.BlockSpec((B, 1), lambda i: (i, 0)),
          pl.BlockSpec((B, 1), lambda i: (i, 0)),
          pl.BlockSpec((16, 32), lambda i: (0, 0)),
          pl.BlockSpec((1, 32), lambda i: (0, 0)),
          pl.BlockSpec((1, 32), lambda i: (0, 0)),
          pl.BlockSpec((32, 1152), lambda i: (0, 0)),
      ],
      out_specs=[
          pl.BlockSpec((B, 32), lambda i: (i, 0)),
          pl.BlockSpec((9, B, 128), lambda i: (0, i, 0)),
      ],
      out_shape=[
          jax.ShapeDtypeStruct((N0, 32), jnp.float32),
          jax.ShapeDtypeStruct((9, N0, 128), jnp.float32),
      ],
  )(a16, x2d, dg, w1f, r1, b1, w2f)


def _t2_call(s2, h1, dg, r2, b2, w3f):
  B = 512

  def fn(s_r, h_r, d_r, r2_r, b2_r, w3_r, hp_o, t3_o):
    d = jnp.maximum(d_r[...], 1.0)
    h = s_r[...] / d + jnp.dot(h_r[...], r2_r[...],
                               preferred_element_type=jnp.float32) + b2_r[...]
    h = _elu(h)
    hp = h.reshape(B // 2, 2, 32).max(axis=1)
    hp_o[...] = hp
    t = jnp.dot(hp, w3_r[...], preferred_element_type=jnp.float32)
    t3_o[...] = t.reshape(B // 2, 9, 256).transpose(1, 0, 2)

  return pl.pallas_call(
      fn,
      grid=(N0 // B,),
      in_specs=[
          pl.BlockSpec((B, 32), lambda i: (i, 0)),
          pl.BlockSpec((B, 32), lambda i: (i, 0)),
          pl.BlockSpec((B, 1), lambda i: (i, 0)),
          pl.BlockSpec((32, 32), lambda i: (0, 0)),
          pl.BlockSpec((1, 32), lambda i: (0, 0)),
          pl.BlockSpec((32, 2304), lambda i: (0, 0)),
      ],
      out_specs=[
          pl.BlockSpec((B // 2, 32), lambda i: (i, 0)),
          pl.BlockSpec((9, B // 2, 256), lambda i: (0, i, 0)),
      ],
      out_shape=[
          jax.ShapeDtypeStruct((N1, 32), jnp.float32),
          jax.ShapeDtypeStruct((9, N1, 256), jnp.float32),
      ],
  )(s2, h1, dg, r2, b2, w3f)


def _t3_call(s3, h2p, dg1, r3, b3, w4f):
  B = 512

  def fn(s_r, h_r, d_r, r3_r, b3_r, w4_r, h3_o, t4_o):
    d = jnp.maximum(d_r[...], 1.0)
    h = s_r[...] / d + jnp.dot(h_r[...], r3_r[...],
                               preferred_element_type=jnp.float32) + b3_r[...]
    h = _elu(h)
    h3_o[...] = h
    t = jnp.dot(h, w4_r[...], preferred_element_type=jnp.float32)
    t4_o[...] = t.reshape(B, 9, 256).transpose(1, 0, 2)

  return pl.pallas_call(
      fn,
      grid=(N1 // B,),
      in_specs=[
          pl.BlockSpec((B, 64), lambda i: (i, 0)),
          pl.BlockSpec((B, 32), lambda i: (i, 0)),
          pl.BlockSpec((B, 1), lambda i: (i, 0)),
          pl.BlockSpec((32, 64), lambda i: (0, 0)),
          pl.BlockSpec((1, 64), lambda i: (0, 0)),
          pl.BlockSpec((64, 2304), lambda i: (0, 0)),
      ],
      out_specs=[
          pl.BlockSpec((B, 64), lambda i: (i, 0)),
          pl.BlockSpec((9, B, 256), lambda i: (0, i, 0)),
      ],
      out_shape=[
          jax.ShapeDtypeStruct((N1, 64), jnp.float32),
          jax.ShapeDtypeStruct((9, N1, 256), jnp.float32),
      ],
  )(s3, h2p, dg1, r3, b3, w4f)


def _t4_call(s4, h3, dg1, r4, b4):
  B = 512

  def fn(s_r, h_r, d_r, r4_r, b4_r, hp_o):
    d = jnp.maximum(d_r[...], 1.0)
    h = s_r[...] / d + jnp.dot(h_r[...], r4_r[...],
                               preferred_element_type=jnp.float32) + b4_r[...]
    h = _elu(h)
    hp_o[...] = h.reshape(B // 2, 2, 64).max(axis=1)

  return pl.pallas_call(
      fn,
      grid=(N1 // B,),
      in_specs=[
          pl.BlockSpec((B, 64), lambda i: (i, 0)),
          pl.BlockSpec((B, 64), lambda i: (i, 0)),
          pl.BlockSpec((B, 1), lambda i: (i, 0)),
          pl.BlockSpec((64, 64), lambda i: (0, 0)),
          pl.BlockSpec((1, 64), lambda i: (0, 0)),
      ],
      out_specs=pl.BlockSpec((B // 2, 64), lambda i: (i, 0)),
      out_shape=jax.ShapeDtypeStruct((N1 // 2, 64), jnp.float32),
  )(s4, h3, dg1, r4, b4)


def _t5_call(xfc, fc1_w, fc1_b):
  KB = 64
  BK = 65536 // KB

  def fn(x_r, w_r, b_r, o_r, acc_r):
    k = pl.program_id(0)

    @pl.when(k == 0)
    def _():
      acc_r[...] = jnp.zeros_like(acc_r)

    acc_r[...] += jnp.dot(x_r[...], w_r[...],
                          preferred_element_type=jnp.float32)

    @pl.when(k == KB - 1)
    def _():
      o_r[...] = _elu(acc_r[...] + b_r[...])

  return pl.pallas_call(
      fn,
      grid=(KB,),
      in_specs=[
          pl.BlockSpec((16, BK), lambda k: (0, k)),
          pl.BlockSpec((BK, 512), lambda k: (k, 0)),
          pl.BlockSpec((1, 512), lambda k: (0, 0)),
      ],
      out_specs=pl.BlockSpec((16, 512), lambda k: (0, 0)),
      out_shape=jax.ShapeDtypeStruct((16, 512), jnp.float32),
      scratch_shapes=[pltpu.VMEM((16, 512), jnp.float32)],
  )(xfc, fc1_w, fc1_b)


def _t6_call(h5, fc2_w, fc2_b):
  def fn(h_r, w_r, b_r, o_r):
    z = jnp.dot(h_r[...], w_r[...],
                preferred_element_type=jnp.float32) + b_r[...]
    m = jnp.max(z, axis=-1, keepdims=True)
    e = jnp.exp(z - m)
    s = jnp.sum(e, axis=-1, keepdims=True)
    o_r[...] = z - m - jnp.log(s)

  return pl.pallas_call(
      fn,
      out_shape=jax.ShapeDtypeStruct((16, 10), jnp.float32),
  )(h5, fc2_w, fc2_b)


_a1_l0 = _make_a1(E0, 11)
_a2_l0 = _make_a2(N0, E0, 11, 16)
_conv1_l0 = _make_conv1(N0, E0, 11, 16)
_conv2_l0 = _make_conv_t(N0, E0, 11, 16, 32, 64)
_a1_l1 = _make_a1(E1, 10)
_a2_l1 = _make_a2(N1, E1, 10, 15)
_deg_l1 = _make_deg(N1, E1)
_conv3_l1 = _make_conv_t(N1, E1, 10, 15, 64, 32)
_conv4_l1 = _make_conv_t(N1, E1, 10, 15, 64, 32)

_KIDX9 = tuple((i0s + c0) * 4 + (i1s + c1)
               for i0s in range(3) for i1s in range(3)
               for (c0, c1) in ((0, 0), (0, 1), (1, 0), (1, 1)))


def _combo_w(W, cin, cout):
  wf = W.transpose(1, 0, 2)
  wf = wf[:, _KIDX9, :]
  return wf.reshape(cin, 36 * cout)


def kernel(x, edge_index0, pseudo0, edge_index1, pseudo1,
           W1, root1, b1, W2, root2, b2, W3, root3, b3, W4, root4, b4,
           fc1_w, fc1_b, fc2_w, fc2_b):
  src0 = edge_index0[0]
  dst0 = edge_index0[1]
  ps0 = pseudo0.reshape(-1)
  src1 = edge_index1[0]
  dst1 = edge_index1[1]
  ps1 = pseudo1.reshape(-1)

  hist0 = _a1_l0(dst0)
  rsk0, rf00, rf10, rdst0, boff0 = _a2_l0(src0, dst0, ps0, hist0)
  a16, dg0 = _conv1_l0(rsk0, rf00, rf10, rdst0, boff0, x[:, 0])
  a16 = a16.reshape(N0, 16)
  dg0c = dg0.reshape(N0, 1)

  w1f = W1.reshape(16, 32)
  w2f9 = _combo_w(W2, 32, 32)
  h1, t2 = _t1_call(a16, x, dg0c, w1f, root1.reshape(1, 32),
                    b1.reshape(1, 32), w2f9)
  s2 = _conv2_l0(rsk0, rf00, rf10, rdst0, boff0,
                 t2.reshape(9 * N0, 128)).reshape(N0, 32)

  w3f9 = _combo_w(W3, 32, 64)
  h2p, t3 = _t2_call(s2, h1, dg0c, root2, b2.reshape(1, 32), w3f9)

  hist1 = _a1_l1(dst1)
  rsk1, rf01, rf11, rdst1, boff1 = _a2_l1(src1, dst1, ps1, hist1)
  dg1 = _deg_l1(rdst1, boff1)
  dg1c = dg1.reshape(N1, 1)

  s3 = _conv3_l1(rsk1, rf01, rf11, rdst1, boff1,
                 t3.reshape(9 * N1, 256)).reshape(N1, 64)
  w4f9 = _combo_w(W4, 64, 64)
  h3, t4 = _t3_call(s3, h2p, dg1c, root3, b3.reshape(1, 64), w4f9)
  s4 = _conv4_l1(rsk1, rf01, rf11, rdst1, boff1,
                 t4.reshape(9 * N1, 256)).reshape(N1, 64)
  h4p = _t4_call(s4, h3, dg1c, root4, b4.reshape(1, 64))

  xfc = h4p.reshape(16, 65536)
  h5 = _t5_call(xfc, fc1_w, fc1_b.reshape(1, 512))
  return _t6_call(h5, fc2_w, fc2_b.reshape(1, 10))
